# Initial kernel scaffold; baseline (speedup 1.0000x reference)
import jax
import jax.numpy as jnp
from jax.experimental import pallas as pl


def kernel(anchor, positive):
    raise NotImplementedError("write your pallas kernel here")


# speedup vs baseline: 2.2869x; 2.2869x over previous
import functools

import jax
import jax.numpy as jnp
from jax.experimental import pallas as pl

_MARGIN = 1.0
_BM = 512


def _triplet_kernel(a_ref, p_ref, pblk_ref, out_ref):
    i = pl.program_id(0)
    nsteps = pl.num_programs(0)
    a = a_ref[...]
    p = p_ref[...]

    a_norm = jnp.sqrt(jnp.sum(a * a, axis=1, keepdims=True))
    p_norm = jnp.sqrt(jnp.sum(p * p, axis=1, keepdims=True))
    a_n = a / a_norm
    p_n = p / p_norm

    s = jnp.dot(a_n, p_n.T, preferred_element_type=jnp.float32)

    bm, b = s.shape
    row_g = i * bm + jax.lax.broadcasted_iota(jnp.int32, (bm, b), 0)
    col = jax.lax.broadcasted_iota(jnp.int32, (bm, b), 1)
    s = jnp.where(row_g == col, jnp.inf, s)
    an_dist = jnp.min(s, axis=1)

    p_blk = pblk_ref[...]
    pn_blk = jnp.sqrt(jnp.sum(p_blk * p_blk, axis=1, keepdims=True))
    ap_dot = jnp.sum(a * p_blk, axis=1, keepdims=True)
    ap_dist = (ap_dot / jnp.maximum(a_norm * pn_blk, 1e-8))[:, 0]

    partial = jnp.sum(jax.nn.relu(_MARGIN + ap_dist - an_dist)).reshape(1, 1)

    @pl.when(i == 0)
    def _init():
        out_ref[...] = jnp.zeros((1, 1), jnp.float32)

    out_ref[...] += partial

    @pl.when(i == nsteps - 1)
    def _fin():
        out_ref[...] = out_ref[...] / (nsteps * bm)


@functools.partial(jax.jit, static_argnames=("interpret",))
def kernel(anchor, positive, interpret=False):
    B, D = positive.shape
    grid = B // _BM
    out = pl.pallas_call(
        _triplet_kernel,
        grid=(grid,),
        in_specs=[
            pl.BlockSpec((_BM, D), lambda i: (i, 0)),
            pl.BlockSpec((B, D), lambda i: (0, 0)),
            pl.BlockSpec((_BM, D), lambda i: (i, 0)),
        ],
        out_specs=pl.BlockSpec((1, 1), lambda i: (0, 0)),
        out_shape=jax.ShapeDtypeStruct((1, 1), jnp.float32),
        interpret=interpret,
    )(anchor, positive, positive)
    return out[0, 0]


# speedup vs baseline: 3.4592x; 1.5126x over previous
import functools

import jax
import jax.numpy as jnp
from jax.experimental import pallas as pl

_MARGIN = 1.0
_BN = 512


def _triplet_kernel(a_ref, p_ref, out_ref):
    a = a_ref[...]
    p = p_ref[...]
    b, _ = a.shape

    a_norm = jnp.sqrt(jnp.sum(a * a, axis=1, keepdims=True))
    p_norm = jnp.sqrt(jnp.sum(p * p, axis=1, keepdims=True))
    a_n = a / a_norm
    p_n = p / p_norm

    eye = (jax.lax.broadcasted_iota(jnp.int32, (_BN, _BN), 0)
           == jax.lax.broadcasted_iota(jnp.int32, (_BN, _BN), 1))

    acc = jnp.full((b, 1), jnp.inf, jnp.float32)
    for j in range(b // _BN):
        lo, hi = j * _BN, (j + 1) * _BN
        tile = jnp.dot(a_n, p_n[lo:hi, :].T,
                       preferred_element_type=jnp.float32)
        m = jnp.min(tile, axis=1, keepdims=True)
        sub = jnp.where(eye, jnp.inf, tile[lo:hi, :])
        m_sub = jnp.min(sub, axis=1, keepdims=True)
        pieces = ([m[:lo]] if lo else []) + [m_sub] + ([m[hi:]] if hi < b else [])
        m = jnp.concatenate(pieces, axis=0) if len(pieces) > 1 else m_sub
        acc = jnp.minimum(acc, m)

    ap_dot = jnp.sum(a * p, axis=1, keepdims=True)
    ap_dist = ap_dot / jnp.maximum(a_norm * p_norm, 1e-8)

    loss = jnp.sum(jax.nn.relu(_MARGIN + ap_dist - acc)) / b
    out_ref[...] = loss.reshape(1, 1)


@functools.partial(jax.jit, static_argnames=("interpret",))
def kernel(anchor, positive, interpret=False):
    out = pl.pallas_call(
        _triplet_kernel,
        out_shape=jax.ShapeDtypeStruct((1, 1), jnp.float32),
        interpret=interpret,
    )(anchor, positive)
    return out[0, 0]
